# baseline (speedup 1.0000x reference)
import jax
import jax.numpy as jnp
from jax import lax
from jax.experimental import pallas as pl
from jax.experimental.pallas import tpu as pltpu
from jax.experimental.pallas import tpu_sc as plsc

B = 4096
D = 15360
NUM_EMB = 5
NC = 2
NS = 16
NW = NC * NS
BPW = B // NW
W = 32


def _body(x_hbm, table_hbm, out_hbm, idx_v, table_l, sem):
    wid = lax.axis_index("s") * NC + lax.axis_index("c")
    base = wid * BPW
    pltpu.sync_copy(x_hbm.at[wid], idx_v)
    pltpu.sync_copy(table_hbm, table_l)

    def start(r):
        vvec = idx_v[pl.ds((r // 16) * 16, 16)]
        v = vvec[r % 16]
        pltpu.async_copy(table_l.at[v], out_hbm.at[base + r], sem)

    def wait_one():
        pltpu.make_async_copy(table_l.at[0], out_hbm.at[base], sem).wait()

    for r in range(W):
        start(r)
    for r in range(W, BPW):
        wait_one()
        start(r)
    for _ in range(W):
        wait_one()


def _gather(x, table_p):
    f = pl.kernel(
        _body,
        out_type=jax.ShapeDtypeStruct((B, 30, 2, 2, 128), jnp.float32),
        compiler_params=pltpu.CompilerParams(use_tc_tiling_on_sc=True),
        mesh=plsc.VectorSubcoreMesh(core_axis_name="c", subcore_axis_name="s"),
        scratch_types=[
            pltpu.VMEM((BPW,), jnp.int32),
            pltpu.VMEM((NUM_EMB, 30, 2, 2, 128), jnp.float32),
            pltpu.SemaphoreType.DMA,
        ],
    )
    return f(x, table_p)


def kernel(x, table):
    table_p = (
        table.reshape(NUM_EMB, 128, 120)
        .transpose(0, 2, 1)
        .reshape(NUM_EMB, 30, 2, 2, 128)
    )
    out_p = _gather(x.astype(jnp.int32).reshape(NW, BPW), table_p)
    return out_p.transpose(0, 4, 1, 2, 3)


# speedup vs baseline: 1.0091x; 1.0091x over previous
import jax
import jax.numpy as jnp
from jax import lax
from jax.experimental import pallas as pl
from jax.experimental.pallas import tpu as pltpu
from jax.experimental.pallas import tpu_sc as plsc

B = 4096
D = 15360
NUM_EMB = 5
NC = 2
NS = 16
NW = NC * NS
BPW = B // NW
W = 16


def _body(x_hbm, table_hbm, out_hbm, idx_v, table_l, sem):
    wid = lax.axis_index("s") * NC + lax.axis_index("c")
    base = wid * BPW
    pltpu.sync_copy(x_hbm.at[wid], idx_v)
    pltpu.sync_copy(table_hbm, table_l)

    def start(r):
        vvec = idx_v[pl.ds((r // 16) * 16, 16)]
        v = vvec[r % 16]
        pltpu.async_copy(table_l.at[v], out_hbm.at[base + r], sem)

    def wait_one():
        pltpu.make_async_copy(table_l.at[0], out_hbm.at[base], sem).wait()

    for r in range(W):
        start(r)
    for r in range(W, BPW):
        wait_one()
        start(r)
    for _ in range(W):
        wait_one()


def _gather(x, table_p):
    f = pl.kernel(
        _body,
        out_type=jax.ShapeDtypeStruct((B, 30, 2, 2, 128), jnp.float32),
        compiler_params=pltpu.CompilerParams(use_tc_tiling_on_sc=True),
        mesh=plsc.VectorSubcoreMesh(core_axis_name="c", subcore_axis_name="s"),
        scratch_types=[
            pltpu.VMEM((BPW,), jnp.int32),
            pltpu.VMEM((NUM_EMB, 30, 2, 2, 128), jnp.float32),
            pltpu.SemaphoreType.DMA,
        ],
    )
    return f(x, table_p)


def kernel(x, table):
    table_p = (
        table.reshape(NUM_EMB, 128, 120)
        .transpose(0, 2, 1)
        .reshape(NUM_EMB, 30, 2, 2, 128)
    )
    out_p = _gather(x.astype(jnp.int32).reshape(NW, BPW), table_p)
    return out_p.transpose(0, 4, 1, 2, 3)
